# baseline (speedup 1.0000x reference)
import functools

import jax
import jax.numpy as jnp
from jax import lax
from jax.experimental import pallas as pl
from jax.experimental.pallas import tpu as pltpu
from jax.experimental.pallas import tpu_sc as plsc

D_MODEL = 64
NUM_CORES = 2
NUM_SUBCORES = 16
NUM_WORKERS = NUM_CORES * NUM_SUBCORES
C = 256
S_LEN = 200
B_LEN = 4096
CPB = B_LEN // C


def _emb_body(qpw, idx_hbm, tab_hbm, out_hbm, idx_v, rv, rows_v, stage, gsem):
    wid = lax.axis_index("s") * NUM_CORES + lax.axis_index("c")
    lane = lax.iota(jnp.int32, 16)

    def chunk_body(j, carry):
        q = wid * qpw + j
        srow = q // CPB
        b0 = (q % CPB) * C
        off = q * C
        pltpu.sync_copy(idx_hbm.at[pl.ds(off, C)], idx_v)

        def shift_body(i, carry2):
            v = idx_v[pl.ds(i * 16, 16)]
            rv[pl.ds(i * 16, 16)] = lax.shift_right_logical(v, 1)
            return carry2

        lax.fori_loop(0, C // 16, shift_body, 0)
        pltpu.async_copy(tab_hbm.at[rv], rows_v, gsem).wait()

        def grp_body(g, carry2):
            t0 = g * 16
            toks = idx_v[pl.ds(t0, 16)]
            rowi = lane + t0
            colbase = (toks & 1) * 64

            def d_body(d, carry3):
                vals = plsc.load_gather(rows_v, [rowi, colbase + d])
                stage[d, pl.ds(t0, 16)] = vals
                return carry3

            lax.fori_loop(0, D_MODEL, d_body, 0)
            return carry2

        lax.fori_loop(0, C // 16, grp_body, 0)
        pltpu.sync_copy(stage, out_hbm.at[srow, :, pl.ds(b0, C)])
        return carry

    lax.fori_loop(0, qpw, chunk_body, 0)


def kernel(token_ids, emb_table):
    b, s = token_ids.shape
    flat_idx = token_ids.T.reshape(-1).astype(jnp.int32)
    n = flat_idx.shape[0]
    packed = emb_table.reshape(-1, 128)
    nchunks = n // C
    assert nchunks % NUM_WORKERS == 0
    qpw = nchunks // NUM_WORKERS

    mesh = plsc.VectorSubcoreMesh(core_axis_name="c", subcore_axis_name="s")
    k = pl.kernel(
        functools.partial(_emb_body, qpw),
        mesh=mesh,
        out_type=jax.ShapeDtypeStruct((s, D_MODEL, b), jnp.float32),
        scratch_types=[
            pltpu.VMEM((C,), jnp.int32),
            pltpu.VMEM((C,), jnp.int32),
            pltpu.VMEM((C, 128), jnp.float32),
            pltpu.VMEM((D_MODEL, C), jnp.float32),
            pltpu.SemaphoreType.DMA,
        ],
        compiler_params=pltpu.CompilerParams(needs_layout_passes=False),
    )
    out = k(flat_idx, packed)
    return jnp.transpose(out, (2, 0, 1))


# speedup vs baseline: 1.1672x; 1.1672x over previous
import functools

import jax
import jax.numpy as jnp
from jax import lax
from jax.experimental import pallas as pl
from jax.experimental.pallas import tpu as pltpu
from jax.experimental.pallas import tpu_sc as plsc

D_MODEL = 64
NUM_CORES = 2
NUM_SUBCORES = 16
NUM_WORKERS = NUM_CORES * NUM_SUBCORES
C = 256
S_LEN = 200
B_LEN = 4096
CPB = B_LEN // C
ROWS_BYTES = C * 128 * 4
STAGE_BYTES = D_MODEL * C * 4


def _emb_body(qpw, idx_hbm, tab_hbm, out_hbm,
              idx_all, rv0, rv1, rows0, rows1, stage0, stage1,
              gsem0, gsem1, wsem0, wsem1):
    wid = lax.axis_index("s") * NUM_CORES + lax.axis_index("c")
    base = wid * qpw
    lane = lax.iota(jnp.int32, 16)
    rv = (rv0, rv1)
    rows = (rows0, rows1)
    stage = (stage0, stage1)
    gsem = (gsem0, gsem1)
    wsem = (wsem0, wsem1)

    pltpu.sync_copy(idx_hbm.at[pl.ds(base * C, qpw * C)], idx_all)

    def _wb_wait(buf):
        pltpu.make_async_copy(
            stage[buf], out_hbm.at[0, :, pl.ds(0, C)], wsem[buf]).wait()

    def fire(j, buf):
        off = j * C

        def shift_body(i, carry2):
            v = idx_all[pl.ds(off + i * 16, 16)]
            rv[buf][pl.ds(i * 16, 16)] = lax.shift_right_logical(v, 1)
            return carry2

        lax.fori_loop(0, C // 16, shift_body, 0)
        pltpu.async_copy(tab_hbm.at[rv[buf]], rows[buf], gsem[buf])

    def process(j, buf):
        pltpu.make_async_copy(tab_hbm.at[rv[buf]], rows[buf], gsem[buf]).wait()
        off = j * C

        def grp_body(g, carry2):
            t0 = g * 16
            toks = idx_all[pl.ds(off + t0, 16)]
            rowi = lane + t0
            colbase = (toks & 1) * 64
            for d in range(D_MODEL):
                stage[buf][d, pl.ds(t0, 16)] = plsc.load_gather(
                    rows[buf], [rowi, colbase + d])
            return carry2

        lax.fori_loop(0, C // 16, grp_body, 0)
        q = base + j
        srow = q // CPB
        b0 = (q % CPB) * C
        pltpu.async_copy(stage[buf], out_hbm.at[srow, :, pl.ds(b0, C)],
                         wsem[buf])

    fire(0, 0)

    def pair_body(jj, carry):
        j0 = jj * 2

        @pl.when(j0 + 1 < qpw)
        def _():
            fire(j0 + 1, 1)

        @pl.when(jj > 0)
        def _():
            _wb_wait(0)

        process(j0, 0)

        @pl.when(j0 + 2 < qpw)
        def _():
            fire(j0 + 2, 0)

        @pl.when(j0 + 1 < qpw)
        def _():
            @pl.when(jj > 0)
            def _():
                _wb_wait(1)

            process(j0 + 1, 1)

        return carry

    lax.fori_loop(0, (qpw + 1) // 2, pair_body, 0)
    _wb_wait(0)
    _wb_wait(1)


def kernel(token_ids, emb_table):
    b, s = token_ids.shape
    flat_idx = token_ids.T.reshape(-1).astype(jnp.int32)
    n = flat_idx.shape[0]
    packed = emb_table.reshape(-1, 128)
    nchunks = n // C
    assert nchunks % NUM_WORKERS == 0
    qpw = nchunks // NUM_WORKERS

    mesh = plsc.VectorSubcoreMesh(core_axis_name="c", subcore_axis_name="s")
    k = pl.kernel(
        functools.partial(_emb_body, qpw),
        mesh=mesh,
        out_type=jax.ShapeDtypeStruct((s, D_MODEL, b), jnp.float32),
        scratch_types=[
            pltpu.VMEM((qpw * C,), jnp.int32),
            pltpu.VMEM((C,), jnp.int32),
            pltpu.VMEM((C,), jnp.int32),
            pltpu.VMEM((C, 128), jnp.float32),
            pltpu.VMEM((C, 128), jnp.float32),
            pltpu.VMEM((D_MODEL, C), jnp.float32),
            pltpu.VMEM((D_MODEL, C), jnp.float32),
            pltpu.SemaphoreType.DMA,
            pltpu.SemaphoreType.DMA,
            pltpu.SemaphoreType.DMA,
            pltpu.SemaphoreType.DMA,
        ],
        compiler_params=pltpu.CompilerParams(needs_layout_passes=False),
    )
    out = k(flat_idx, packed)
    return jnp.transpose(out, (2, 0, 1))


# speedup vs baseline: 1.5559x; 1.3330x over previous
import functools

import jax
import jax.numpy as jnp
from jax import lax
from jax.experimental import pallas as pl
from jax.experimental.pallas import tpu as pltpu
from jax.experimental.pallas import tpu_sc as plsc

D_MODEL = 64
NUM_CORES = 2
NUM_SUBCORES = 16
NUM_WORKERS = NUM_CORES * NUM_SUBCORES
C = 256
S_LEN = 200
B_LEN = 4096
CPB = B_LEN // C
ROWS_BYTES = C * 128 * 4
STAGE_BYTES = D_MODEL * C * 4


def _emb_body(qpw, idx_hbm, tab_hbm, out_hbm,
              idx_all, rv0, rv1, rows0, rows1, stage0, stage1,
              gsem0, gsem1, wsem0, wsem1):
    wid = lax.axis_index("s") * NUM_CORES + lax.axis_index("c")
    base = wid * qpw
    lane = lax.iota(jnp.int32, 16)
    rv = (rv0, rv1)
    rows = (rows0, rows1)
    stage = (stage0, stage1)
    gsem = (gsem0, gsem1)
    wsem = (wsem0, wsem1)

    pltpu.sync_copy(idx_hbm.at[pl.ds(base * C, qpw * C)], idx_all)

    def _wb_wait(buf):
        pltpu.make_async_copy(
            stage[buf], out_hbm.at[0, :, pl.ds(0, C)], wsem[buf]).wait()

    def fire(j, buf):
        off = j * C

        def shift_body(i, carry2):
            v = idx_all[pl.ds(off + i * 16, 16)]
            rv[buf][pl.ds(i * 16, 16)] = lax.shift_right_logical(v, 1)
            return carry2

        lax.fori_loop(0, C // 16, shift_body, 0)
        pltpu.async_copy(tab_hbm.at[rv[buf]], rows[buf], gsem[buf])

    def process(j, buf):
        pltpu.make_async_copy(tab_hbm.at[rv[buf]], rows[buf], gsem[buf]).wait()
        off = j * C

        @plsc.parallel_loop(0, C // 16, unroll=2)
        def grp_body(g):
            t0 = g * 16
            toks = idx_all[pl.ds(off + t0, 16)]
            rowi = lane + t0
            colbase = (toks & 1) * 64
            for d in range(D_MODEL):
                stage[buf][d, pl.ds(t0, 16)] = plsc.load_gather(
                    rows[buf], [rowi, colbase + d])
        q = base + j
        srow = q // CPB
        b0 = (q % CPB) * C
        pltpu.async_copy(stage[buf], out_hbm.at[srow, :, pl.ds(b0, C)],
                         wsem[buf])

    fire(0, 0)

    def pair_body(jj, carry):
        j0 = jj * 2

        @pl.when(j0 + 1 < qpw)
        def _():
            fire(j0 + 1, 1)

        @pl.when(jj > 0)
        def _():
            _wb_wait(0)

        process(j0, 0)

        @pl.when(j0 + 2 < qpw)
        def _():
            fire(j0 + 2, 0)

        @pl.when(j0 + 1 < qpw)
        def _():
            @pl.when(jj > 0)
            def _():
                _wb_wait(1)

            process(j0 + 1, 1)

        return carry

    lax.fori_loop(0, (qpw + 1) // 2, pair_body, 0)
    _wb_wait(0)
    _wb_wait(1)


def kernel(token_ids, emb_table):
    b, s = token_ids.shape
    flat_idx = token_ids.T.reshape(-1).astype(jnp.int32)
    n = flat_idx.shape[0]
    packed = emb_table.reshape(-1, 128)
    nchunks = n // C
    assert nchunks % NUM_WORKERS == 0
    qpw = nchunks // NUM_WORKERS

    mesh = plsc.VectorSubcoreMesh(core_axis_name="c", subcore_axis_name="s")
    k = pl.kernel(
        functools.partial(_emb_body, qpw),
        mesh=mesh,
        out_type=jax.ShapeDtypeStruct((s, D_MODEL, b), jnp.float32),
        scratch_types=[
            pltpu.VMEM((qpw * C,), jnp.int32),
            pltpu.VMEM((C,), jnp.int32),
            pltpu.VMEM((C,), jnp.int32),
            pltpu.VMEM((C, 128), jnp.float32),
            pltpu.VMEM((C, 128), jnp.float32),
            pltpu.VMEM((D_MODEL, C), jnp.float32),
            pltpu.VMEM((D_MODEL, C), jnp.float32),
            pltpu.SemaphoreType.DMA,
            pltpu.SemaphoreType.DMA,
            pltpu.SemaphoreType.DMA,
            pltpu.SemaphoreType.DMA,
        ],
        compiler_params=pltpu.CompilerParams(needs_layout_passes=False),
    )
    out = k(flat_idx, packed)
    return jnp.transpose(out, (2, 0, 1))


# speedup vs baseline: 1.6847x; 1.0828x over previous
import functools

import jax
import jax.numpy as jnp
from jax import lax
from jax.experimental import pallas as pl
from jax.experimental.pallas import tpu as pltpu
from jax.experimental.pallas import tpu_sc as plsc

D_MODEL = 64
NUM_CORES = 2
NUM_SUBCORES = 16
NUM_WORKERS = NUM_CORES * NUM_SUBCORES
C = 256
S_LEN = 200
B_LEN = 4096
CPB = B_LEN // C
ROWS_BYTES = C * 128 * 4
STAGE_BYTES = D_MODEL * C * 4


def _emb_body(qpw, idx_hbm, tab_hbm, out_hbm,
              idx_all, rv0, rv1, rows0, rows1, stage0, stage1,
              gsem0, gsem1, wsem0, wsem1):
    wid = lax.axis_index("s") * NUM_CORES + lax.axis_index("c")
    base = wid * qpw
    lane = lax.iota(jnp.int32, 16)
    rv = (rv0, rv1)
    rows = (rows0, rows1)
    stage = (stage0, stage1)
    gsem = (gsem0, gsem1)
    wsem = (wsem0, wsem1)

    pltpu.sync_copy(idx_hbm.at[pl.ds(base * C, qpw * C)], idx_all)

    def _wb_wait(buf):
        pltpu.make_async_copy(
            stage[buf], out_hbm.at[0, :, pl.ds(0, C)], wsem[buf]).wait()

    def fire(j, buf):
        off = j * C

        def shift_body(i, carry2):
            v = idx_all[pl.ds(off + i * 16, 16)]
            rv[buf][pl.ds(i * 16, 16)] = lax.shift_right_logical(v, 1)
            return carry2

        lax.fori_loop(0, C // 16, shift_body, 0)
        pltpu.async_copy(tab_hbm.at[rv[buf]], rows[buf], gsem[buf])

    def process(j, buf):
        pltpu.make_async_copy(tab_hbm.at[rv[buf]], rows[buf], gsem[buf]).wait()
        off = j * C

        def grp_body(g, carry2):
            t0 = g * 16
            toks = idx_all[pl.ds(off + t0, 16)]
            rowi = lane + t0
            colbase = (toks & 1) * 64

            @plsc.parallel_loop(0, D_MODEL, unroll=8)
            def d_body(d):
                stage[buf][d, pl.ds(t0, 16)] = plsc.load_gather(
                    rows[buf], [rowi, colbase + d])

            return carry2

        lax.fori_loop(0, C // 16, grp_body, 0)
        q = base + j
        srow = q // CPB
        b0 = (q % CPB) * C
        pltpu.async_copy(stage[buf], out_hbm.at[srow, :, pl.ds(b0, C)],
                         wsem[buf])

    fire(0, 0)

    def pair_body(jj, carry):
        j0 = jj * 2

        @pl.when(j0 + 1 < qpw)
        def _():
            fire(j0 + 1, 1)

        @pl.when(jj > 0)
        def _():
            _wb_wait(0)

        process(j0, 0)

        @pl.when(j0 + 2 < qpw)
        def _():
            fire(j0 + 2, 0)

        @pl.when(j0 + 1 < qpw)
        def _():
            @pl.when(jj > 0)
            def _():
                _wb_wait(1)

            process(j0 + 1, 1)

        return carry

    lax.fori_loop(0, (qpw + 1) // 2, pair_body, 0)
    _wb_wait(0)
    _wb_wait(1)


def kernel(token_ids, emb_table):
    b, s = token_ids.shape
    flat_idx = token_ids.T.reshape(-1).astype(jnp.int32)
    n = flat_idx.shape[0]
    packed = emb_table.reshape(-1, 128)
    nchunks = n // C
    assert nchunks % NUM_WORKERS == 0
    qpw = nchunks // NUM_WORKERS

    mesh = plsc.VectorSubcoreMesh(core_axis_name="c", subcore_axis_name="s")
    k = pl.kernel(
        functools.partial(_emb_body, qpw),
        mesh=mesh,
        out_type=jax.ShapeDtypeStruct((s, D_MODEL, b), jnp.float32),
        scratch_types=[
            pltpu.VMEM((qpw * C,), jnp.int32),
            pltpu.VMEM((C,), jnp.int32),
            pltpu.VMEM((C,), jnp.int32),
            pltpu.VMEM((C, 128), jnp.float32),
            pltpu.VMEM((C, 128), jnp.float32),
            pltpu.VMEM((D_MODEL, C), jnp.float32),
            pltpu.VMEM((D_MODEL, C), jnp.float32),
            pltpu.SemaphoreType.DMA,
            pltpu.SemaphoreType.DMA,
            pltpu.SemaphoreType.DMA,
            pltpu.SemaphoreType.DMA,
        ],
        compiler_params=pltpu.CompilerParams(needs_layout_passes=False),
    )
    out = k(flat_idx, packed)
    return jnp.transpose(out, (2, 0, 1))


# speedup vs baseline: 2.3774x; 1.4112x over previous
import functools

import jax
import jax.numpy as jnp
from jax import lax
from jax.experimental import pallas as pl
from jax.experimental.pallas import tpu as pltpu
from jax.experimental.pallas import tpu_sc as plsc

D_MODEL = 64
NUM_CORES = 2
NUM_SUBCORES = 16
NUM_WORKERS = NUM_CORES * NUM_SUBCORES
C = 256
S_LEN = 200
B_LEN = 4096
CPB = B_LEN // C
ROWS_BYTES = C * 128 * 4
STAGE_BYTES = D_MODEL * C * 4


def _emb_body(qpw, idx_hbm, tab_hbm, out_hbm,
              idx_all, rv0, rv1, rows0, rows1, stage0, stage1,
              gsem0, gsem1, wsem0, wsem1):
    wid = lax.axis_index("s") * NUM_CORES + lax.axis_index("c")
    base = wid * qpw
    lane = lax.iota(jnp.int32, 16)
    wraps = [(lane + j) & 15 for j in range(16)]
    rv = (rv0, rv1)
    rows = (rows0, rows1)
    stage = (stage0, stage1)
    gsem = (gsem0, gsem1)
    wsem = (wsem0, wsem1)

    pltpu.sync_copy(idx_hbm.at[pl.ds(base * C, qpw * C)], idx_all)

    def _wb_wait(buf):
        pltpu.make_async_copy(
            stage[buf], out_hbm.at[0, :, pl.ds(0, C)], wsem[buf]).wait()

    def fire(j, buf):
        off = j * C

        def shift_body(i, carry2):
            v = idx_all[pl.ds(off + i * 16, 16)]
            rv[buf][pl.ds(i * 16, 16)] = lax.shift_right_logical(v, 1)
            return carry2

        lax.fori_loop(0, C // 16, shift_body, 0)
        pltpu.async_copy(tab_hbm.at[rv[buf]], rows[buf], gsem[buf])

    def process(j, buf):
        pltpu.make_async_copy(tab_hbm.at[rv[buf]], rows[buf], gsem[buf]).wait()
        off = j * C

        @plsc.parallel_loop(0, C // 16, unroll=2)
        def grp_body(g):
            t0 = g * 16
            toks = idx_all[pl.ds(off + t0, 16)]
            rowi = lane + t0
            pbase = (toks & 1) * 64
            for k in range(D_MODEL // 16):
                for j in range(16):
                    w = wraps[j] + (16 * k)
                    vals = plsc.load_gather(rows[buf], [rowi, pbase + w])
                    plsc.store_scatter(stage[buf], [w, rowi], vals)
        q = base + j
        srow = q // CPB
        b0 = (q % CPB) * C
        pltpu.async_copy(stage[buf], out_hbm.at[srow, :, pl.ds(b0, C)],
                         wsem[buf])

    fire(0, 0)

    def pair_body(jj, carry):
        j0 = jj * 2

        @pl.when(j0 + 1 < qpw)
        def _():
            fire(j0 + 1, 1)

        @pl.when(jj > 0)
        def _():
            _wb_wait(0)

        process(j0, 0)

        @pl.when(j0 + 2 < qpw)
        def _():
            fire(j0 + 2, 0)

        @pl.when(j0 + 1 < qpw)
        def _():
            @pl.when(jj > 0)
            def _():
                _wb_wait(1)

            process(j0 + 1, 1)

        return carry

    lax.fori_loop(0, (qpw + 1) // 2, pair_body, 0)
    _wb_wait(0)
    _wb_wait(1)


def kernel(token_ids, emb_table):
    b, s = token_ids.shape
    flat_idx = token_ids.T.reshape(-1).astype(jnp.int32)
    n = flat_idx.shape[0]
    packed = emb_table.reshape(-1, 128)
    nchunks = n // C
    assert nchunks % NUM_WORKERS == 0
    qpw = nchunks // NUM_WORKERS

    mesh = plsc.VectorSubcoreMesh(core_axis_name="c", subcore_axis_name="s")
    k = pl.kernel(
        functools.partial(_emb_body, qpw),
        mesh=mesh,
        out_type=jax.ShapeDtypeStruct((s, D_MODEL, b), jnp.float32),
        scratch_types=[
            pltpu.VMEM((qpw * C,), jnp.int32),
            pltpu.VMEM((C,), jnp.int32),
            pltpu.VMEM((C,), jnp.int32),
            pltpu.VMEM((C, 128), jnp.float32),
            pltpu.VMEM((C, 128), jnp.float32),
            pltpu.VMEM((D_MODEL, C), jnp.float32),
            pltpu.VMEM((D_MODEL, C), jnp.float32),
            pltpu.SemaphoreType.DMA,
            pltpu.SemaphoreType.DMA,
            pltpu.SemaphoreType.DMA,
            pltpu.SemaphoreType.DMA,
        ],
        compiler_params=pltpu.CompilerParams(needs_layout_passes=False),
    )
    out = k(flat_idx, packed)
    return jnp.transpose(out, (2, 0, 1))


# speedup vs baseline: 3.8579x; 1.6227x over previous
import functools

import jax
import jax.numpy as jnp
from jax import lax
from jax.experimental import pallas as pl
from jax.experimental.pallas import tpu as pltpu
from jax.experimental.pallas import tpu_sc as plsc

D_MODEL = 64
NUM_CORES = 2
NUM_SUBCORES = 16
NUM_WORKERS = NUM_CORES * NUM_SUBCORES
C = 256
S_LEN = 200
B_LEN = 4096
CPB = B_LEN // C
ROWS_BYTES = C * 128 * 4
STAGE_BYTES = D_MODEL * C * 4


def _emb_body(qpw, idx_hbm, tab_hbm, out_hbm,
              idx_all, rv0, rv1, rows0, rows1, stage0, stage1,
              gsem0, gsem1, wsem0, wsem1):
    wid = lax.axis_index("s") * NUM_CORES + lax.axis_index("c")
    base = wid * qpw
    lane = lax.iota(jnp.int32, 16)
    wraps = [(lane + j) & 15 for j in range(16)]
    rv = (rv0, rv1)
    rows = (rows0, rows1)
    stage = (stage0, stage1)
    gsem = (gsem0, gsem1)
    wsem = (wsem0, wsem1)

    pltpu.sync_copy(idx_hbm.at[pl.ds(base * C, qpw * C)], idx_all)

    def _wb_wait(buf):
        pltpu.make_async_copy(
            stage[buf], out_hbm.at[0, :, pl.ds(0, C)], wsem[buf]).wait()

    def fire(j, buf):
        off = j * C

        def shift_body(i, carry2):
            v = idx_all[pl.ds(off + i * 16, 16)]
            rv[buf][pl.ds(i * 16, 16)] = lax.shift_right_logical(v, 1)
            return carry2

        lax.fori_loop(0, C // 16, shift_body, 0)
        pltpu.async_copy(tab_hbm.at[rv[buf]], rows[buf], gsem[buf])

    def process(j, buf):
        pltpu.make_async_copy(tab_hbm.at[rv[buf]], rows[buf], gsem[buf]).wait()
        off = j * C

        @plsc.parallel_loop(0, C // 16, unroll=2)
        def grp_body(g):
            t0 = g * 16
            toks = idx_all[pl.ds(off + t0, 16)]
            rowi = lane + t0
            pbase = (toks & 1) * 64
            for k in range(D_MODEL // 16):
                for j in range(16):
                    w = wraps[j] + (16 * k)
                    vals = plsc.load_gather(rows[buf], [rowi, pbase + w])
                    plsc.store_scatter(stage[buf], [w, rowi], vals)
        q = base + j
        srow = q // CPB
        b0 = (q % CPB) * C
        pltpu.async_copy(stage[buf], out_hbm.at[srow, :, pl.ds(b0, C)],
                         wsem[buf])

    fire(0, 0)

    def pair_body(jj, carry):
        j0 = jj * 2

        @pl.when(j0 + 1 < qpw)
        def _():
            fire(j0 + 1, 1)

        @pl.when(jj > 0)
        def _():
            _wb_wait(0)

        process(j0, 0)

        @pl.when(j0 + 2 < qpw)
        def _():
            fire(j0 + 2, 0)

        @pl.when(j0 + 1 < qpw)
        def _():
            @pl.when(jj > 0)
            def _():
                _wb_wait(1)

            process(j0 + 1, 1)

        return carry

    lax.fori_loop(0, (qpw + 1) // 2, pair_body, 0)
    _wb_wait(0)
    _wb_wait(1)


PK_FULL = 7812
PK_BPW = (PK_FULL + NUM_WORKERS - 1) // NUM_WORKERS
PK_TAIL_C0 = PK_FULL * 128
BIN_BYTES = 64 * 128 * 4
BOUT_BYTES = 64 * 128 * 4


def _pack_body(tabT_hbm, out_hbm, bin0, bin1, bout0, bout1,
               isem0, isem1, osem0, osem1):
    wid = lax.axis_index("s") * NUM_CORES + lax.axis_index("c")
    lane = lax.iota(jnp.int32, 16)
    wraps = [(lane + j) & 15 for j in range(16)]
    bins = (bin0, bin1)
    bouts = (bout0, bout1)
    isem = (isem0, isem1)
    osem = (osem0, osem1)
    b_lo = wid * PK_BPW
    nblk = jnp.minimum(PK_BPW, jnp.maximum(PK_FULL - b_lo, 0))

    def fire_in(t, buf):
        c0 = (b_lo + t) * 128
        pltpu.async_copy(tabT_hbm.at[:, pl.ds(c0, 128)], bins[buf], isem[buf])

    def wait_wb(buf):
        pltpu.make_async_copy(
            bouts[buf], out_hbm.at[pl.ds(0, 64), :], osem[buf]).wait()

    def transpose_block(binb, boutb, tgroups):
        @plsc.parallel_loop(0, tgroups * 8, unroll=2)
        def m_body(m):
            tb = m >> 3
            p = (m >> 2) & 1
            k = m & 3
            trow = 16 * tb + lane
            for j in range(16):
                w = wraps[j] + 16 * k
                vals = plsc.load_gather(binb, [w, 2 * trow + p])
                plsc.store_scatter(boutb, [trow, 64 * p + w], vals)

    def process(t, buf):
        bid = b_lo + t
        pltpu.make_async_copy(
            tabT_hbm.at[:, pl.ds(0, 128)], bins[buf], isem[buf]).wait()
        transpose_block(bins[buf], bouts[buf], 4)
        pltpu.async_copy(bouts[buf], out_hbm.at[pl.ds(64 * bid, 64), :],
                         osem[buf])

    @pl.when(nblk > 0)
    def _():
        fire_in(0, 0)

    def pair_body(jj, carry):
        j0 = jj * 2

        @pl.when(j0 + 1 < nblk)
        def _():
            fire_in(j0 + 1, 1)

        @pl.when(jj > 0)
        def _():
            wait_wb(0)

        process(j0, 0)

        @pl.when(j0 + 2 < nblk)
        def _():
            fire_in(j0 + 2, 0)

        @pl.when(j0 + 1 < nblk)
        def _():
            @pl.when(jj > 0)
            def _():
                wait_wb(1)

            process(j0 + 1, 1)

        return carry

    lax.fori_loop(0, (nblk + 1) // 2, pair_body, 0)

    @pl.when(nblk > 0)
    def _():
        wait_wb(0)

    @pl.when(nblk > 1)
    def _():
        wait_wb(1)

    @pl.when(wid == NUM_WORKERS - 1)
    def _():
        for d in range(64):
            pltpu.sync_copy(tabT_hbm.at[d, pl.ds(PK_TAIL_C0, 64)],
                            bin0.at[d, pl.ds(0, 64)])
        transpose_block(bin0, bout0, 2)
        pltpu.sync_copy(bout0.at[pl.ds(0, 32), :],
                        out_hbm.at[pl.ds(PK_FULL * 64, 32), :])


def kernel(token_ids, emb_table):
    b, s = token_ids.shape
    flat_idx = token_ids.T.reshape(-1).astype(jnp.int32)
    n = flat_idx.shape[0]
    v = emb_table.shape[0]

    mesh = plsc.VectorSubcoreMesh(core_axis_name="c", subcore_axis_name="s")
    pack_k = pl.kernel(
        _pack_body,
        mesh=mesh,
        out_type=jax.ShapeDtypeStruct((v // 2, 128), jnp.float32),
        scratch_types=[
            pltpu.VMEM((64, 128), jnp.float32),
            pltpu.VMEM((64, 128), jnp.float32),
            pltpu.VMEM((64, 128), jnp.float32),
            pltpu.VMEM((64, 128), jnp.float32),
            pltpu.SemaphoreType.DMA,
            pltpu.SemaphoreType.DMA,
            pltpu.SemaphoreType.DMA,
            pltpu.SemaphoreType.DMA,
        ],
        compiler_params=pltpu.CompilerParams(needs_layout_passes=False),
    )
    packed = pack_k(emb_table.T)
    nchunks = n // C
    assert nchunks % NUM_WORKERS == 0
    qpw = nchunks // NUM_WORKERS

    k = pl.kernel(
        functools.partial(_emb_body, qpw),
        mesh=mesh,
        out_type=jax.ShapeDtypeStruct((s, D_MODEL, b), jnp.float32),
        scratch_types=[
            pltpu.VMEM((qpw * C,), jnp.int32),
            pltpu.VMEM((C,), jnp.int32),
            pltpu.VMEM((C,), jnp.int32),
            pltpu.VMEM((C, 128), jnp.float32),
            pltpu.VMEM((C, 128), jnp.float32),
            pltpu.VMEM((D_MODEL, C), jnp.float32),
            pltpu.VMEM((D_MODEL, C), jnp.float32),
            pltpu.SemaphoreType.DMA,
            pltpu.SemaphoreType.DMA,
            pltpu.SemaphoreType.DMA,
            pltpu.SemaphoreType.DMA,
        ],
        compiler_params=pltpu.CompilerParams(needs_layout_passes=False),
    )
    out = k(flat_idx, packed)
    return jnp.transpose(out, (2, 0, 1))


# speedup vs baseline: 4.1467x; 1.0749x over previous
import functools

import jax
import jax.numpy as jnp
from jax import lax
from jax.experimental import pallas as pl
from jax.experimental.pallas import tpu as pltpu
from jax.experimental.pallas import tpu_sc as plsc

D_MODEL = 64
NUM_CORES = 2
NUM_SUBCORES = 16
NUM_WORKERS = NUM_CORES * NUM_SUBCORES
C = 256
S_LEN = 200
B_LEN = 4096
CPB = B_LEN // C
ROWS_BYTES = C * 128 * 4
STAGE_BYTES = D_MODEL * C * 4


def _emb_body(qpw, idx_hbm, tab_hbm, out_hbm,
              idx_all, rv0, rv1, rows0, rows1, stage0, stage1,
              gsem0, gsem1, wsem0, wsem1):
    wid = lax.axis_index("s") * NUM_CORES + lax.axis_index("c")
    base = wid * qpw
    lane = lax.iota(jnp.int32, 16)
    wraps = [(lane + j) & 15 for j in range(16)]
    rv = (rv0, rv1)
    rows = (rows0, rows1)
    stage = (stage0, stage1)
    gsem = (gsem0, gsem1)
    wsem = (wsem0, wsem1)

    pltpu.sync_copy(idx_hbm.at[pl.ds(base * C, qpw * C)], idx_all)

    def _wb_wait(buf):
        pltpu.make_async_copy(
            stage[buf], out_hbm.at[0, :, pl.ds(0, C)], wsem[buf]).wait()

    def fire(j, buf):
        off = j * C

        def shift_body(i, carry2):
            v = idx_all[pl.ds(off + i * 16, 16)]
            rv[buf][pl.ds(i * 16, 16)] = lax.shift_right_logical(v, 1)
            return carry2

        lax.fori_loop(0, C // 16, shift_body, 0)
        pltpu.async_copy(tab_hbm.at[rv[buf]], rows[buf], gsem[buf])

    def process(j, buf):
        pltpu.make_async_copy(tab_hbm.at[rv[buf]], rows[buf], gsem[buf]).wait()
        off = j * C

        @plsc.parallel_loop(0, C // 16, unroll=2)
        def grp_body(g):
            t0 = g * 16
            toks = idx_all[pl.ds(off + t0, 16)]
            rowi = lane + t0
            pbase = (toks & 1) * 64
            for k in range(D_MODEL // 16):
                for j in range(16):
                    w = wraps[j] + (16 * k)
                    vals = plsc.load_gather(rows[buf], [rowi, pbase + w])
                    plsc.store_scatter(stage[buf], [w, rowi], vals)
        q = base + j
        srow = q // CPB
        b0 = (q % CPB) * C
        pltpu.async_copy(stage[buf], out_hbm.at[srow, :, pl.ds(b0, C)],
                         wsem[buf])

    fire(0, 0)

    def pair_body(jj, carry):
        j0 = jj * 2

        @pl.when(j0 + 1 < qpw)
        def _():
            fire(j0 + 1, 1)

        @pl.when(jj > 0)
        def _():
            _wb_wait(0)

        process(j0, 0)

        @pl.when(j0 + 2 < qpw)
        def _():
            fire(j0 + 2, 0)

        @pl.when(j0 + 1 < qpw)
        def _():
            @pl.when(jj > 0)
            def _():
                _wb_wait(1)

            process(j0 + 1, 1)

        return carry

    lax.fori_loop(0, (qpw + 1) // 2, pair_body, 0)
    _wb_wait(0)
    _wb_wait(1)


PK_BC = 256
PK_FULL = 3906
PK_BPW = (PK_FULL + NUM_WORKERS - 1) // NUM_WORKERS
PK_TAIL_C0 = PK_FULL * PK_BC


def _pack_body(tabT_hbm, out_hbm, bin0, bin1, bout0, bout1,
               isem0, isem1, osem0, osem1):
    wid = lax.axis_index("s") * NUM_CORES + lax.axis_index("c")
    lane = lax.iota(jnp.int32, 16)
    wraps = [(lane + j) & 15 for j in range(16)]
    bins = (bin0, bin1)
    bouts = (bout0, bout1)
    isem = (isem0, isem1)
    osem = (osem0, osem1)
    b_lo = wid * PK_BPW
    nblk = jnp.minimum(PK_BPW, jnp.maximum(PK_FULL - b_lo, 0))

    def fire_in(t, buf):
        c0 = (b_lo + t) * PK_BC
        pltpu.async_copy(tabT_hbm.at[:, pl.ds(c0, PK_BC)], bins[buf], isem[buf])

    def wait_wb(buf):
        pltpu.make_async_copy(
            bouts[buf], out_hbm.at[pl.ds(0, PK_BC // 2), :], osem[buf]).wait()

    def transpose_block(binb, boutb, tgroups):
        @plsc.parallel_loop(0, tgroups * 8, unroll=2)
        def m_body(m):
            tb = m >> 3
            p = (m >> 2) & 1
            k = m & 3
            trow = 16 * tb + lane
            for j in range(16):
                w = wraps[j] + 16 * k
                vals = plsc.load_gather(binb, [w, 2 * trow + p])
                plsc.store_scatter(boutb, [trow, 64 * p + w], vals)

    def process(t, buf):
        bid = b_lo + t
        pltpu.make_async_copy(
            tabT_hbm.at[:, pl.ds(0, PK_BC)], bins[buf], isem[buf]).wait()
        transpose_block(bins[buf], bouts[buf], PK_BC // 32)
        pltpu.async_copy(
            bouts[buf],
            out_hbm.at[pl.ds((PK_BC // 2) * bid, PK_BC // 2), :],
            osem[buf])

    @pl.when(nblk > 0)
    def _():
        fire_in(0, 0)

    def pair_body(jj, carry):
        j0 = jj * 2

        @pl.when(j0 + 1 < nblk)
        def _():
            fire_in(j0 + 1, 1)

        @pl.when(jj > 0)
        def _():
            wait_wb(0)

        process(j0, 0)

        @pl.when(j0 + 2 < nblk)
        def _():
            fire_in(j0 + 2, 0)

        @pl.when(j0 + 1 < nblk)
        def _():
            @pl.when(jj > 0)
            def _():
                wait_wb(1)

            process(j0 + 1, 1)

        return carry

    lax.fori_loop(0, (nblk + 1) // 2, pair_body, 0)

    @pl.when(nblk > 0)
    def _():
        wait_wb(0)

    @pl.when(nblk > 1)
    def _():
        wait_wb(1)

    @pl.when(wid == NUM_WORKERS - 1)
    def _():
        for d in range(64):
            pltpu.sync_copy(tabT_hbm.at[d, pl.ds(PK_TAIL_C0, 64)],
                            bin0.at[d, pl.ds(0, 64)])
        transpose_block(bin0, bout0, 2)
        pltpu.sync_copy(bout0.at[pl.ds(0, 32), :],
                        out_hbm.at[pl.ds(PK_TAIL_C0 // 2, 32), :])


def kernel(token_ids, emb_table):
    b, s = token_ids.shape
    flat_idx = token_ids.T.reshape(-1).astype(jnp.int32)
    n = flat_idx.shape[0]
    v = emb_table.shape[0]

    mesh = plsc.VectorSubcoreMesh(core_axis_name="c", subcore_axis_name="s")
    pack_k = pl.kernel(
        _pack_body,
        mesh=mesh,
        out_type=jax.ShapeDtypeStruct((v // 2, 128), jnp.float32),
        scratch_types=[
            pltpu.VMEM((64, PK_BC), jnp.float32),
            pltpu.VMEM((64, PK_BC), jnp.float32),
            pltpu.VMEM((PK_BC // 2, 128), jnp.float32),
            pltpu.VMEM((PK_BC // 2, 128), jnp.float32),
            pltpu.SemaphoreType.DMA,
            pltpu.SemaphoreType.DMA,
            pltpu.SemaphoreType.DMA,
            pltpu.SemaphoreType.DMA,
        ],
        compiler_params=pltpu.CompilerParams(needs_layout_passes=False),
    )
    packed = pack_k(emb_table.T)
    nchunks = n // C
    assert nchunks % NUM_WORKERS == 0
    qpw = nchunks // NUM_WORKERS

    k = pl.kernel(
        functools.partial(_emb_body, qpw),
        mesh=mesh,
        out_type=jax.ShapeDtypeStruct((s, D_MODEL, b), jnp.float32),
        scratch_types=[
            pltpu.VMEM((qpw * C,), jnp.int32),
            pltpu.VMEM((C,), jnp.int32),
            pltpu.VMEM((C,), jnp.int32),
            pltpu.VMEM((C, 128), jnp.float32),
            pltpu.VMEM((C, 128), jnp.float32),
            pltpu.VMEM((D_MODEL, C), jnp.float32),
            pltpu.VMEM((D_MODEL, C), jnp.float32),
            pltpu.SemaphoreType.DMA,
            pltpu.SemaphoreType.DMA,
            pltpu.SemaphoreType.DMA,
            pltpu.SemaphoreType.DMA,
        ],
        compiler_params=pltpu.CompilerParams(needs_layout_passes=False),
    )
    out = k(flat_idx, packed)
    return jnp.transpose(out, (2, 0, 1))
